# Initial kernel scaffold; baseline (speedup 1.0000x reference)
import jax
import jax.numpy as jnp
from jax.experimental import pallas as pl


def kernel(x, word_embed, idf, fc1_w, fc1_b, fc2_w, fc2_b):
    raise NotImplementedError("write your pallas kernel here")


# speedup vs baseline: 4.8590x; 4.8590x over previous
import functools

import jax
import jax.numpy as jnp
from jax import lax
from jax.experimental import pallas as pl
from jax.experimental.pallas import tpu as pltpu
from jax.experimental.pallas import tpu_sc as plsc

NC, NS = 2, 16
NW = NC * NS
CH = 128
KCH = 5


def _sc_gather(table, idx_flat):
    tok = idx_flat.shape[0]
    dp = table.shape[1]
    per_w = tok // NW
    niter = per_w // (CH * KCH)
    mesh = plsc.VectorSubcoreMesh(core_axis_name="c", subcore_axis_name="s")
    scratch = ([pltpu.VMEM((per_w,), jnp.int32)]
               + [pltpu.VMEM((CH, dp), jnp.float32) for _ in range(KCH)]
               + [pltpu.SemaphoreType.DMA for _ in range(KCH)]
               + [pltpu.SemaphoreType.DMA for _ in range(KCH)])

    @functools.partial(pl.kernel,
                       out_type=jax.ShapeDtypeStruct((tok, dp), jnp.float32),
                       mesh=mesh, scratch_types=scratch)
    def run(tab_ref, idx_ref, out_ref, idx_v, *rest):
        bufs = rest[:KCH]
        gsems = rest[KCH:2 * KCH]
        osems = rest[2 * KCH:3 * KCH]
        wid = lax.axis_index("s") * NC + lax.axis_index("c")
        base = wid * per_w
        pltpu.sync_copy(idx_ref.at[pl.ds(base, per_w)], idx_v)

        @pl.loop(0, niter)
        def _(k):
            c0 = k * KCH
            gcp = [pltpu.async_copy(
                tab_ref.at[idx_v.at[pl.ds(pl.multiple_of((c0 + i) * CH, CH), CH)]],
                bufs[i], gsems[i]) for i in range(KCH)]
            ocp = []
            for i in range(KCH):
                gcp[i].wait()
                off = pl.multiple_of(base + (c0 + i) * CH, CH)
                ocp.append(pltpu.async_copy(bufs[i], out_ref.at[pl.ds(off, CH)],
                                            osems[i]))
            for cp in ocp:
                cp.wait()

    return run(table, idx_flat)


def _tc_counts(x):
    b, l = x.shape
    db = 8

    def body(x_ref, o_ref):
        xb = x_ref[...]
        eq = (xb[:, :, None] == xb[:, None, :]).astype(jnp.float32)
        o_ref[...] = jnp.sum(eq, axis=2).reshape(db * l, 1)

    return pl.pallas_call(
        body,
        grid=(b // db,),
        in_specs=[pl.BlockSpec((db, l), lambda i: (i, 0))],
        out_specs=pl.BlockSpec((db * l, 1), lambda i: (i, 0)),
        out_shape=jax.ShapeDtypeStruct((b * l, 1), jnp.float32),
    )(x)


def _tc_reduce(g, cnt, fc1_w, fc1_b, fc2_w, fc2_b, b, l, d):
    dp = g.shape[1]
    db = 32
    r = db * l

    def body(g_ref, c_ref, w1_ref, b1_ref, w2_ref, b2_ref, o_ref):
        gb = g_ref[...]
        idf_col = gb[:, d:d + 1]
        denom = jnp.sum(idf_col.reshape(db, l, 1), axis=1, keepdims=True)
        wcol = (c_ref[...].reshape(db, l, 1) * idf_col.reshape(db, l, 1)
                / denom).reshape(r, 1)
        ws = jnp.sum((gb * wcol).reshape(db, l, dp), axis=1)
        h = lax.dot_general(ws[:, :d], w1_ref[...], (((1,), (1,)), ((), ())),
                            precision=lax.Precision.HIGHEST) + b1_ref[...]
        o_ref[...] = lax.dot_general(h, w2_ref[...], (((1,), (1,)), ((), ())),
                                     precision=lax.Precision.HIGHEST) + b2_ref[...]

    return pl.pallas_call(
        body,
        grid=(b // db,),
        in_specs=[pl.BlockSpec((r, dp), lambda i: (i, 0)),
                  pl.BlockSpec((r, 1), lambda i: (i, 0)),
                  pl.BlockSpec(fc1_w.shape, lambda i: (0, 0)),
                  pl.BlockSpec((1, fc1_b.shape[0]), lambda i: (0, 0)),
                  pl.BlockSpec(fc2_w.shape, lambda i: (0, 0)),
                  pl.BlockSpec((1, fc2_b.shape[0]), lambda i: (0, 0))],
        out_specs=pl.BlockSpec((db, d), lambda i: (i, 0)),
        out_shape=jax.ShapeDtypeStruct((b, d), jnp.float32),
    )(g, cnt, fc1_w, fc1_b.reshape(1, -1), fc2_w, fc2_b.reshape(1, -1))


def kernel(x, word_embed, idf, fc1_w, fc1_b, fc2_w, fc2_b):
    v, d = word_embed.shape
    b, l = x.shape
    dp = 128
    table = jnp.concatenate(
        [word_embed, idf[:, None], jnp.zeros((v, dp - d - 1), jnp.float32)],
        axis=1)
    g = _sc_gather(table, x.reshape(-1))
    cnt = _tc_counts(x)
    return _tc_reduce(g, cnt, fc1_w, fc1_b, fc2_w, fc2_b, b, l, d)
